# baseline (speedup 1.0000x reference)
import dataclasses
import functools

import jax
import jax.numpy as jnp
from jax import lax
from jax.experimental import pallas as pl
from jax.experimental.pallas import tpu as pltpu
from jax.experimental.pallas import tpu_sc as plsc

B = 16384
F = 64
NC = 2
NS = 16
NW = NC * NS
BPW = B // NW
L = 16
RND = 2
CPR = BPW // RND


def _compiler_params():
    cp = pltpu.CompilerParams(use_tc_tiling_on_sc=True)
    if "needs_layout_passes" in pltpu.CompilerParams.__dataclass_fields__:
        cp = dataclasses.replace(cp, needs_layout_passes=False)
    return cp


def _pack_pairs(table_t):
    n = table_t.shape[1]
    blk = 512
    grid = (n + 2 * blk - 1) // (2 * blk)

    def body(in_a, in_b, out_ref):
        out_ref[:, 0:F] = in_a[...].T
        out_ref[:, F:2 * F] = in_b[...].T

    return pl.pallas_call(
        body,
        grid=(grid,),
        in_specs=[
            pl.BlockSpec((F, blk), lambda i: (0, 2 * i)),
            pl.BlockSpec((F, blk), lambda i: (0, 2 * i + 1)),
        ],
        out_specs=pl.BlockSpec((blk, 2 * F), lambda i: (i, 0)),
        out_shape=jax.ShapeDtypeStruct((grid * blk, 2 * F), jnp.float32),
    )(table_t, table_t)


def kernel(cell_indices, drug_indices, cell_factors, drug_factors,
           cell_bias, drug_bias, global_bias):
    cfp = _pack_pairs(cell_factors.T)
    dfp = _pack_pairs(drug_factors.T)
    cell_bias_flat = cell_bias.reshape(-1)
    drug_bias_flat = drug_bias.reshape(-1)
    global_bias16 = jnp.broadcast_to(global_bias, (L,))
    mesh = plsc.VectorSubcoreMesh(core_axis_name="c", subcore_axis_name="s")

    @functools.partial(
        pl.kernel,
        out_type=jax.ShapeDtypeStruct((B,), jnp.float32),
        mesh=mesh,
        compiler_params=_compiler_params(),
        scratch_types=[
            pltpu.VMEM((BPW,), jnp.int32),
            pltpu.VMEM((BPW,), jnp.int32),
            pltpu.VMEM((CPR,), jnp.int32),
            pltpu.VMEM((CPR,), jnp.int32),
            pltpu.VMEM((CPR, 2 * F), jnp.float32),
            pltpu.VMEM((CPR, 2 * F), jnp.float32),
            pltpu.VMEM((BPW,), jnp.float32),
            pltpu.VMEM((BPW,), jnp.float32),
            pltpu.VMEM((BPW,), jnp.float32),
            pltpu.VMEM((L,), jnp.float32),
            pltpu.SemaphoreType.DMA,
            pltpu.SemaphoreType.DMA,
            pltpu.SemaphoreType.DMA,
        ],
    )
    def sc_kernel(ci_hbm, di_hbm, cfp_hbm, dfp_hbm, cb_hbm, db_hbm, gb_hbm,
                  out_hbm, ci_v, di_v, pci_v, pdi_v, cr_v, dr_v, cb_v, db_v,
                  out_v, gb_v, sem_c, sem_d, sem_m):
        wid = lax.axis_index("s") * NC + lax.axis_index("c")
        base = wid * BPW

        pltpu.sync_copy(ci_hbm.at[pl.ds(base, BPW)], ci_v)
        pltpu.sync_copy(di_hbm.at[pl.ds(base, BPW)], di_v)
        pltpu.sync_copy(gb_hbm, gb_v)

        h_cb = pltpu.async_copy(cb_hbm.at[ci_v], cb_v, sem_m)
        h_db = pltpu.async_copy(db_hbm.at[di_v], db_v, sem_m)
        h_cb.wait()
        h_db.wait()

        g = gb_v[...]
        lanes = lax.iota(jnp.int32, L)

        for rnd in range(RND):
            r_off = rnd * CPR

            @pl.loop(0, CPR, step=L)
            def _(j):
                ci = ci_v[pl.ds(r_off + j, L)]
                di = di_v[pl.ds(r_off + j, L)]
                pci_v[pl.ds(j, L)] = (
                    lax.shift_left(lax.shift_right_logical(ci, 10), 9)
                    + (ci & 511))
                pdi_v[pl.ds(j, L)] = (
                    lax.shift_left(lax.shift_right_logical(di, 10), 9)
                    + (di & 511))

            h_c = pltpu.async_copy(cfp_hbm.at[pci_v], cr_v, sem_c)
            h_d = pltpu.async_copy(dfp_hbm.at[pdi_v], dr_v, sem_d)
            h_c.wait()
            h_d.wait()

            @pl.loop(0, CPR, step=L)
            def _(j):
                rows = j + lanes
                colc = lax.shift_left(
                    lax.shift_right_logical(ci_v[pl.ds(r_off + j, L)], 9) & 1,
                    6)
                cold = lax.shift_left(
                    lax.shift_right_logical(di_v[pl.ds(r_off + j, L)], 9) & 1,
                    6)
                acc0 = (cb_v[pl.ds(r_off + j, L)] +
                        db_v[pl.ds(r_off + j, L)] + g)
                acc1 = jnp.zeros((L,), jnp.float32)
                acc2 = jnp.zeros((L,), jnp.float32)
                acc3 = jnp.zeros((L,), jnp.float32)
                accs = [acc0, acc1, acc2, acc3]
                for f in range(F):
                    cg = plsc.load_gather(cr_v, [rows, colc + f])
                    dg = plsc.load_gather(dr_v, [rows, cold + f])
                    accs[f % 4] = accs[f % 4] + cg * dg
                out_v[pl.ds(r_off + j, L)] = ((accs[0] + accs[1]) +
                                              (accs[2] + accs[3]))

        pltpu.sync_copy(out_v, out_hbm.at[pl.ds(base, BPW)])

    return sc_kernel(cell_indices, drug_indices, cfp, dfp,
                     cell_bias_flat, drug_bias_flat, global_bias16)


# speedup vs baseline: 1.9886x; 1.9886x over previous
import dataclasses
import functools

import jax
import jax.numpy as jnp
from jax import lax
from jax.experimental import pallas as pl
from jax.experimental.pallas import tpu as pltpu
from jax.experimental.pallas import tpu_sc as plsc

B = 16384
F = 64
NC = 2
NS = 16
NW = NC * NS
BPW = B // NW
L = 16
RND = 2
CPR = BPW // RND
PBLK = 2048
PSH_HI = 12
PSH_LO = 11
PMASK = PBLK - 1


def _compiler_params():
    cp = pltpu.CompilerParams(use_tc_tiling_on_sc=True)
    if "needs_layout_passes" in pltpu.CompilerParams.__dataclass_fields__:
        cp = dataclasses.replace(cp, needs_layout_passes=False)
    return cp


def _pack_pairs(table_t):
    n = table_t.shape[1]
    blk = PBLK
    grid = (n + 2 * blk - 1) // (2 * blk)
    last_blk = (n - 1) // blk

    def body(in_a, in_b, out_ref):
        row = lax.broadcasted_iota(jnp.int32, (F, F), 0)
        col = lax.broadcasted_iota(jnp.int32, (F, F), 1)
        eye = (row == col).astype(jnp.bfloat16)
        dn = (((0,), (0,)), ((), ()))
        out_ref[:, 0:F] = lax.dot_general(
            in_a[...].astype(jnp.bfloat16), eye, dn,
            preferred_element_type=jnp.float32)
        out_ref[:, F:2 * F] = lax.dot_general(
            in_b[...].astype(jnp.bfloat16), eye, dn,
            preferred_element_type=jnp.float32)

    return pl.pallas_call(
        body,
        grid=(grid,),
        in_specs=[
            pl.BlockSpec((F, blk), lambda i: (0, 2 * i)),
            pl.BlockSpec((F, blk),
                         lambda i: (0, jnp.minimum(2 * i + 1, last_blk))),
        ],
        out_specs=pl.BlockSpec((blk, 2 * F), lambda i: (i, 0)),
        out_shape=jax.ShapeDtypeStruct((grid * blk, 2 * F), jnp.float32),
    )(table_t, table_t)


def kernel(cell_indices, drug_indices, cell_factors, drug_factors,
           cell_bias, drug_bias, global_bias):
    cfp = _pack_pairs(cell_factors.T)
    dfp = _pack_pairs(drug_factors.T)
    cell_bias_flat = cell_bias.reshape(-1)
    drug_bias_flat = drug_bias.reshape(-1)
    global_bias16 = jnp.broadcast_to(global_bias, (L,))
    mesh = plsc.VectorSubcoreMesh(core_axis_name="c", subcore_axis_name="s")

    @functools.partial(
        pl.kernel,
        out_type=jax.ShapeDtypeStruct((B,), jnp.float32),
        mesh=mesh,
        compiler_params=_compiler_params(),
        scratch_types=[
            pltpu.VMEM((BPW,), jnp.int32),
            pltpu.VMEM((BPW,), jnp.int32),
            pltpu.VMEM((CPR,), jnp.int32),
            pltpu.VMEM((CPR,), jnp.int32),
            pltpu.VMEM((CPR, 2 * F), jnp.float32),
            pltpu.VMEM((CPR, 2 * F), jnp.float32),
            pltpu.VMEM((BPW,), jnp.float32),
            pltpu.VMEM((BPW,), jnp.float32),
            pltpu.VMEM((BPW,), jnp.float32),
            pltpu.VMEM((L,), jnp.float32),
            pltpu.SemaphoreType.DMA,
            pltpu.SemaphoreType.DMA,
            pltpu.SemaphoreType.DMA,
        ],
    )
    def sc_kernel(ci_hbm, di_hbm, cfp_hbm, dfp_hbm, cb_hbm, db_hbm, gb_hbm,
                  out_hbm, ci_v, di_v, pci_v, pdi_v, cr_v, dr_v, cb_v, db_v,
                  out_v, gb_v, sem_c, sem_d, sem_m):
        wid = lax.axis_index("s") * NC + lax.axis_index("c")
        base = wid * BPW

        pltpu.sync_copy(ci_hbm.at[pl.ds(base, BPW)], ci_v)
        pltpu.sync_copy(di_hbm.at[pl.ds(base, BPW)], di_v)
        pltpu.sync_copy(gb_hbm, gb_v)

        h_cb = pltpu.async_copy(cb_hbm.at[ci_v], cb_v, sem_m)
        h_db = pltpu.async_copy(db_hbm.at[di_v], db_v, sem_m)
        h_cb.wait()
        h_db.wait()

        g = gb_v[...]
        lanes = lax.iota(jnp.int32, L)

        for rnd in range(RND):
            r_off = rnd * CPR

            @pl.loop(0, CPR, step=L)
            def _(j):
                ci = ci_v[pl.ds(r_off + j, L)]
                di = di_v[pl.ds(r_off + j, L)]
                pci_v[pl.ds(j, L)] = (
                    lax.shift_left(lax.shift_right_logical(ci, PSH_HI),
                                   PSH_LO) + (ci & PMASK))
                pdi_v[pl.ds(j, L)] = (
                    lax.shift_left(lax.shift_right_logical(di, PSH_HI),
                                   PSH_LO) + (di & PMASK))

            h_c = pltpu.async_copy(cfp_hbm.at[pci_v], cr_v, sem_c)
            h_d = pltpu.async_copy(dfp_hbm.at[pdi_v], dr_v, sem_d)
            h_c.wait()
            h_d.wait()

            @pl.loop(0, CPR, step=L)
            def _(j):
                rows = j + lanes
                colc = lax.shift_left(
                    lax.shift_right_logical(
                        ci_v[pl.ds(r_off + j, L)], PSH_LO) & 1, 6)
                cold = lax.shift_left(
                    lax.shift_right_logical(
                        di_v[pl.ds(r_off + j, L)], PSH_LO) & 1, 6)
                acc0 = (cb_v[pl.ds(r_off + j, L)] +
                        db_v[pl.ds(r_off + j, L)] + g)
                acc1 = jnp.zeros((L,), jnp.float32)
                acc2 = jnp.zeros((L,), jnp.float32)
                acc3 = jnp.zeros((L,), jnp.float32)
                accs = [acc0, acc1, acc2, acc3]
                for f in range(F):
                    cg = plsc.load_gather(cr_v, [rows, colc + f])
                    dg = plsc.load_gather(dr_v, [rows, cold + f])
                    accs[f % 4] = accs[f % 4] + cg * dg
                out_v[pl.ds(r_off + j, L)] = ((accs[0] + accs[1]) +
                                              (accs[2] + accs[3]))

        pltpu.sync_copy(out_v, out_hbm.at[pl.ds(base, BPW)])

    return sc_kernel(cell_indices, drug_indices, cfp, dfp,
                     cell_bias_flat, drug_bias_flat, global_bias16)


# speedup vs baseline: 2.6870x; 1.3512x over previous
import dataclasses
import functools

import jax
import jax.numpy as jnp
from jax import lax
from jax.experimental import pallas as pl
from jax.experimental.pallas import tpu as pltpu
from jax.experimental.pallas import tpu_sc as plsc

B = 16384
F = 64
NC = 2
NS = 16
NW = NC * NS
BPW = B // NW
L = 16
RND = 2
CPR = BPW // RND
PBLK = 8192
PSH_HI = 14
PSH_LO = 13
PMASK = PBLK - 1


def _compiler_params():
    cp = pltpu.CompilerParams(use_tc_tiling_on_sc=True)
    if "needs_layout_passes" in pltpu.CompilerParams.__dataclass_fields__:
        cp = dataclasses.replace(cp, needs_layout_passes=False)
    return cp


def _pack_pairs(table_t):
    n = table_t.shape[1]
    blk = PBLK
    grid = (n + 2 * blk - 1) // (2 * blk)

    def body(in_ref, out_ref):
        row = lax.broadcasted_iota(jnp.int32, (F, F), 0)
        col = lax.broadcasted_iota(jnp.int32, (F, F), 1)
        eye = (row == col).astype(jnp.bfloat16)
        dn = (((0,), (0,)), ((), ()))
        x = in_ref[...].astype(jnp.bfloat16)
        out_ref[:, 0:F] = lax.dot_general(
            x[:, 0:blk], eye, dn, preferred_element_type=jnp.float32)
        out_ref[:, F:2 * F] = lax.dot_general(
            x[:, blk:2 * blk], eye, dn, preferred_element_type=jnp.float32)

    return pl.pallas_call(
        body,
        grid=(grid,),
        in_specs=[pl.BlockSpec((F, 2 * blk), lambda i: (0, i))],
        out_specs=pl.BlockSpec((blk, 2 * F), lambda i: (i, 0)),
        out_shape=jax.ShapeDtypeStruct((grid * blk, 2 * F), jnp.float32),
    )(table_t)


def kernel(cell_indices, drug_indices, cell_factors, drug_factors,
           cell_bias, drug_bias, global_bias):
    cfp = _pack_pairs(cell_factors.T)
    dfp = _pack_pairs(drug_factors.T)
    cell_bias_flat = cell_bias.reshape(-1)
    drug_bias_flat = drug_bias.reshape(-1)
    global_bias16 = jnp.broadcast_to(global_bias, (L,))
    mesh = plsc.VectorSubcoreMesh(core_axis_name="c", subcore_axis_name="s")

    @functools.partial(
        pl.kernel,
        out_type=jax.ShapeDtypeStruct((B,), jnp.float32),
        mesh=mesh,
        compiler_params=_compiler_params(),
        scratch_types=[
            pltpu.VMEM((BPW,), jnp.int32),
            pltpu.VMEM((BPW,), jnp.int32),
            pltpu.VMEM((CPR,), jnp.int32),
            pltpu.VMEM((CPR,), jnp.int32),
            pltpu.VMEM((CPR, 2 * F), jnp.float32),
            pltpu.VMEM((CPR, 2 * F), jnp.float32),
            pltpu.VMEM((BPW,), jnp.float32),
            pltpu.VMEM((BPW,), jnp.float32),
            pltpu.VMEM((BPW,), jnp.float32),
            pltpu.VMEM((L,), jnp.float32),
            pltpu.SemaphoreType.DMA,
            pltpu.SemaphoreType.DMA,
            pltpu.SemaphoreType.DMA,
        ],
    )
    def sc_kernel(ci_hbm, di_hbm, cfp_hbm, dfp_hbm, cb_hbm, db_hbm, gb_hbm,
                  out_hbm, ci_v, di_v, pci_v, pdi_v, cr_v, dr_v, cb_v, db_v,
                  out_v, gb_v, sem_c, sem_d, sem_m):
        wid = lax.axis_index("s") * NC + lax.axis_index("c")
        base = wid * BPW

        pltpu.sync_copy(ci_hbm.at[pl.ds(base, BPW)], ci_v)
        pltpu.sync_copy(di_hbm.at[pl.ds(base, BPW)], di_v)
        pltpu.sync_copy(gb_hbm, gb_v)

        h_cb = pltpu.async_copy(cb_hbm.at[ci_v], cb_v, sem_m)
        h_db = pltpu.async_copy(db_hbm.at[di_v], db_v, sem_m)
        h_cb.wait()
        h_db.wait()

        g = gb_v[...]
        lanes = lax.iota(jnp.int32, L)

        for rnd in range(RND):
            r_off = rnd * CPR

            @pl.loop(0, CPR, step=L)
            def _(j):
                ci = ci_v[pl.ds(r_off + j, L)]
                di = di_v[pl.ds(r_off + j, L)]
                pci_v[pl.ds(j, L)] = (
                    lax.shift_left(lax.shift_right_logical(ci, PSH_HI),
                                   PSH_LO) + (ci & PMASK))
                pdi_v[pl.ds(j, L)] = (
                    lax.shift_left(lax.shift_right_logical(di, PSH_HI),
                                   PSH_LO) + (di & PMASK))

            h_c = pltpu.async_copy(cfp_hbm.at[pci_v], cr_v, sem_c)
            h_d = pltpu.async_copy(dfp_hbm.at[pdi_v], dr_v, sem_d)
            h_c.wait()
            h_d.wait()

            @pl.loop(0, CPR, step=L)
            def _(j):
                rows = j + lanes
                colc = lax.shift_left(
                    lax.shift_right_logical(
                        ci_v[pl.ds(r_off + j, L)], PSH_LO) & 1, 6)
                cold = lax.shift_left(
                    lax.shift_right_logical(
                        di_v[pl.ds(r_off + j, L)], PSH_LO) & 1, 6)
                acc0 = (cb_v[pl.ds(r_off + j, L)] +
                        db_v[pl.ds(r_off + j, L)] + g)
                acc1 = jnp.zeros((L,), jnp.float32)
                acc2 = jnp.zeros((L,), jnp.float32)
                acc3 = jnp.zeros((L,), jnp.float32)
                accs = [acc0, acc1, acc2, acc3]
                for f in range(F):
                    cg = plsc.load_gather(cr_v, [rows, colc + f])
                    dg = plsc.load_gather(dr_v, [rows, cold + f])
                    accs[f % 4] = accs[f % 4] + cg * dg
                out_v[pl.ds(r_off + j, L)] = ((accs[0] + accs[1]) +
                                              (accs[2] + accs[3]))

        pltpu.sync_copy(out_v, out_hbm.at[pl.ds(base, BPW)])

    return sc_kernel(cell_indices, drug_indices, cfp, dfp,
                     cell_bias_flat, drug_bias_flat, global_bias16)
